# baseline (speedup 1.0000x reference)
import jax
import jax.numpy as jnp
from jax.experimental import pallas as pl
from jax.experimental.pallas import tpu as pltpu

B = 1024
D = 2048
NBLK = 1
BLK = B // NBLK
K2 = 16
KC = 4
NEG_INF = -65000.0


def _graph_loss_kernel(fv_ref, fi_ref, labr_ref, labc_ref, out_ref,
                       fvh_s, fih_s, acc_s):
    p = pl.program_id(0)
    i = pl.program_id(1)

    @pl.when(p == 0)
    def _normalize():
        fv = fv_ref[...]
        nv = jnp.sqrt(jnp.sum(fv * fv, axis=1, keepdims=True))
        fvh_s[pl.ds(i * BLK, BLK), :] = (
            fv / jnp.maximum(nv, 1e-12)).astype(jnp.bfloat16)
        fi = fi_ref[...]
        ni = jnp.sqrt(jnp.sum(fi * fi, axis=1, keepdims=True))
        fih_s[pl.ds(i * BLK, BLK), :] = (
            fi / jnp.maximum(ni, 1e-12)).astype(jnp.bfloat16)

        @pl.when(i == 0)
        def _init():
            acc_s[...] = jnp.zeros((1, 2), jnp.float32)

    @pl.when(p == 1)
    def _compute():
        fvh = fvh_s[...]
        fih = fih_s[...]
        avh = fvh_s[pl.ds(i * BLK, BLK), :]
        aih = fih_s[pl.ds(i * BLK, BLK), :]

        dn = (((1,), (1,)), ((), ()))

        def dot(a, b):
            return jax.lax.dot_general(
                a, b, dn, preferred_element_type=jnp.float32)

        simv = dot(avh, fvh)
        simi = dot(aih, fih)
        x = dot(avh, fih)
        y = dot(aih, fvh)

        lab_row = labr_ref[...]
        lab_col = labc_ref[...]
        j_idx = jax.lax.broadcasted_iota(jnp.int32, (BLK, B), 1)
        row_ids = i * BLK + jax.lax.broadcasted_iota(jnp.int32, (BLK, B), 0)
        mask = (lab_col == lab_row) & (j_idx != row_ids)
        rev_j = 1023 - j_idx
        imin = jnp.int32(-2147483648)

        def kth_largest(work, k):
            for _ in range(k - 1):
                m = jnp.max(work, axis=1, keepdims=True)
                work = jnp.where(work == m, imin, work)
            return jnp.max(work, axis=1, keepdims=True)

        def sim_key(sim):
            bits = jax.lax.bitcast_convert_type(sim, jnp.int32)
            skey = bits ^ (jax.lax.shift_right_arithmetic(bits, 31)
                           & jnp.int32(0x7FFFFFFF))
            return jnp.where(mask, (skey & jnp.int32(~1023)) | rev_j, imin)

        keyv = sim_key(simv)
        keyi = sim_key(simi)
        thr2 = kth_largest(jnp.concatenate([keyv, keyi], axis=0), K2)
        inter = ((keyv >= thr2[:BLK]) & (keyi >= thr2[BLK:])) & mask

        ikey = jnp.where(inter, rev_j, -1)
        thr4 = kth_largest(ikey, KC)
        m4f = ((ikey >= thr4) & inter).astype(jnp.float32)
        interf = inter.astype(jnp.float32)

        num = jnp.sum(m4f * x * y, axis=1, keepdims=True)
        na2 = jnp.sum(m4f * x * x, axis=1, keepdims=True)
        nb2 = jnp.sum(m4f * y * y, axis=1, keepdims=True)
        na = jnp.maximum(jnp.sqrt(na2), 1e-8)
        nb = jnp.maximum(jnp.sqrt(nb2), 1e-8)
        contrib = 1.0 - num / (na * nb)
        hasf = (jnp.sum(interf, axis=1, keepdims=True)
                > 0.0).astype(jnp.float32)

        part = jnp.concatenate(
            [jnp.sum(hasf * contrib, axis=0, keepdims=True),
             jnp.sum(hasf, axis=0, keepdims=True)], axis=1)
        acc_s[...] = acc_s[...] + part

        @pl.when(i == NBLK - 1)
        def _finish():
            total = acc_s[0:1, 0:1]
            cnt = acc_s[0:1, 1:2]
            loss = jnp.where(cnt == 0.0, 0.0, total / jnp.maximum(cnt, 1.0))
            loss = jnp.where(jnp.isnan(loss) | jnp.isinf(loss), 0.0, loss)
            out_ref[...] = jnp.maximum(loss, 0.0)


def _run(feat_v, feat_i, lab_row, lab_col, interpret=False):
    return pl.pallas_call(
        _graph_loss_kernel,
        grid=(2, NBLK),
        in_specs=[
            pl.BlockSpec((BLK, D), lambda p, i: (i, 0)),
            pl.BlockSpec((BLK, D), lambda p, i: (i, 0)),
            pl.BlockSpec((1, B), lambda p, i: (0, 0)),
            pl.BlockSpec((BLK, 1), lambda p, i: (i, 0)),
        ],
        out_specs=pl.BlockSpec((1, 1), lambda p, i: (0, 0)),
        out_shape=jax.ShapeDtypeStruct((1, 1), jnp.float32),
        scratch_shapes=[
            pltpu.VMEM((B, D), jnp.bfloat16),
            pltpu.VMEM((B, D), jnp.bfloat16),
            pltpu.VMEM((1, 2), jnp.float32),
        ],
        interpret=interpret,
    )(feat_v, feat_i, lab_row, lab_col)


@jax.jit
def kernel(feat_v, feat_i, labels):
    lab = labels.astype(jnp.int32)
    out = _run(feat_v.astype(jnp.float32), feat_i.astype(jnp.float32),
               lab.reshape(1, B), lab.reshape(B, 1))
    return out[0, 0]


# speedup vs baseline: 1.5970x; 1.5970x over previous
import jax
import jax.numpy as jnp
from jax.experimental import pallas as pl
from jax.experimental.pallas import tpu as pltpu

B = 1024
D = 2048
NBLK = 2
BLK = B // NBLK
K2 = 16
KC = 4
NEG_INF = -65000.0


def _graph_loss_kernel(fv_ref, fi_ref, labr_ref, labc_ref, out_ref,
                       fvh_s, fih_s, acc_s):
    p = pl.program_id(0)
    i = pl.program_id(1)

    @pl.when(p == 0)
    def _normalize():
        fv = fv_ref[...]
        nv = jnp.sqrt(jnp.sum(fv * fv, axis=1, keepdims=True))
        fvh_s[pl.ds(i * BLK, BLK), :] = (
            fv / jnp.maximum(nv, 1e-12)).astype(jnp.bfloat16)
        fi = fi_ref[...]
        ni = jnp.sqrt(jnp.sum(fi * fi, axis=1, keepdims=True))
        fih_s[pl.ds(i * BLK, BLK), :] = (
            fi / jnp.maximum(ni, 1e-12)).astype(jnp.bfloat16)

        @pl.when(i == 0)
        def _init():
            acc_s[...] = jnp.zeros((1, 2), jnp.float32)

    @pl.when(p == 1)
    def _compute():
        fvh = fvh_s[...]
        fih = fih_s[...]
        avh = fvh_s[pl.ds(i * BLK, BLK), :]
        aih = fih_s[pl.ds(i * BLK, BLK), :]

        dn = (((1,), (1,)), ((), ()))

        def dot(a, b):
            return jax.lax.dot_general(
                a, b, dn, preferred_element_type=jnp.float32)

        simv = dot(avh, fvh)
        simi = dot(aih, fih)
        x = dot(avh, fih)
        y = dot(aih, fvh)

        lab_row = labr_ref[...]
        lab_col = labc_ref[...]
        j_idx = jax.lax.broadcasted_iota(jnp.int32, (BLK, B), 1)
        row_ids = i * BLK + jax.lax.broadcasted_iota(jnp.int32, (BLK, B), 0)
        mask = (lab_col == lab_row) & (j_idx != row_ids)
        rev_j = 1023 - j_idx
        imin = jnp.int32(-2147483648)

        def kth_largest(work, k):
            for _ in range(k - 1):
                m = jnp.max(work, axis=1, keepdims=True)
                work = jnp.where(work == m, imin, work)
            return jnp.max(work, axis=1, keepdims=True)

        def sim_key(sim):
            bits = jax.lax.bitcast_convert_type(sim, jnp.int32)
            skey = bits ^ (jax.lax.shift_right_arithmetic(bits, 31)
                           & jnp.int32(0x7FFFFFFF))
            return jnp.where(mask, (skey & jnp.int32(~1023)) | rev_j, imin)

        keyv = sim_key(simv)
        keyi = sim_key(simi)
        thr2 = kth_largest(jnp.concatenate([keyv, keyi], axis=0), K2)
        inter = ((keyv >= thr2[:BLK]) & (keyi >= thr2[BLK:])) & mask

        ikey = jnp.where(inter, rev_j, -1)
        thr4 = kth_largest(ikey, KC)
        m4f = ((ikey >= thr4) & inter).astype(jnp.float32)
        interf = inter.astype(jnp.float32)

        num = jnp.sum(m4f * x * y, axis=1, keepdims=True)
        na2 = jnp.sum(m4f * x * x, axis=1, keepdims=True)
        nb2 = jnp.sum(m4f * y * y, axis=1, keepdims=True)
        na = jnp.maximum(jnp.sqrt(na2), 1e-8)
        nb = jnp.maximum(jnp.sqrt(nb2), 1e-8)
        contrib = 1.0 - num / (na * nb)
        hasf = (jnp.sum(interf, axis=1, keepdims=True)
                > 0.0).astype(jnp.float32)

        part = jnp.concatenate(
            [jnp.sum(hasf * contrib, axis=0, keepdims=True),
             jnp.sum(hasf, axis=0, keepdims=True)], axis=1)
        acc_s[...] = acc_s[...] + part

        @pl.when(i == NBLK - 1)
        def _finish():
            total = acc_s[0:1, 0:1]
            cnt = acc_s[0:1, 1:2]
            loss = jnp.where(cnt == 0.0, 0.0, total / jnp.maximum(cnt, 1.0))
            loss = jnp.where(jnp.isnan(loss) | jnp.isinf(loss), 0.0, loss)
            out_ref[...] = jnp.maximum(loss, 0.0)


def _run(feat_v, feat_i, lab_row, lab_col, interpret=False):
    return pl.pallas_call(
        _graph_loss_kernel,
        grid=(2, NBLK),
        in_specs=[
            pl.BlockSpec((BLK, D), lambda p, i: (i, 0)),
            pl.BlockSpec((BLK, D), lambda p, i: (i, 0)),
            pl.BlockSpec((1, B), lambda p, i: (0, 0)),
            pl.BlockSpec((BLK, 1), lambda p, i: (i, 0)),
        ],
        out_specs=pl.BlockSpec((1, 1), lambda p, i: (0, 0)),
        out_shape=jax.ShapeDtypeStruct((1, 1), jnp.float32),
        scratch_shapes=[
            pltpu.VMEM((B, D), jnp.bfloat16),
            pltpu.VMEM((B, D), jnp.bfloat16),
            pltpu.VMEM((1, 2), jnp.float32),
        ],
        interpret=interpret,
    )(feat_v, feat_i, lab_row, lab_col)


@jax.jit
def kernel(feat_v, feat_i, labels):
    lab = labels.astype(jnp.int32)
    out = _run(feat_v.astype(jnp.float32), feat_i.astype(jnp.float32),
               lab.reshape(1, B), lab.reshape(B, 1))
    return out[0, 0]


# speedup vs baseline: 1.5990x; 1.0013x over previous
import jax
import jax.numpy as jnp
from jax.experimental import pallas as pl
from jax.experimental.pallas import tpu as pltpu

B = 1024
D = 2048
NBLK = 2
BLK = B // NBLK
K2 = 16
KC = 4
NEG_INF = -65000.0


def _graph_loss_kernel(fv_ref, fi_ref, labr_ref, labc_ref, out_ref,
                       fvh_s, fih_s, acc_s):
    p = pl.program_id(0)
    i = pl.program_id(1)

    @pl.when(p == 0)
    def _normalize():
        fv = fv_ref[...]
        nv = jnp.sqrt(jnp.sum(fv * fv, axis=1, keepdims=True))
        fvh_s[pl.ds(i * BLK, BLK), :] = (
            fv / jnp.maximum(nv, 1e-12)).astype(jnp.bfloat16)
        fi = fi_ref[...]
        ni = jnp.sqrt(jnp.sum(fi * fi, axis=1, keepdims=True))
        fih_s[pl.ds(i * BLK, BLK), :] = (
            fi / jnp.maximum(ni, 1e-12)).astype(jnp.bfloat16)

        @pl.when(i == 0)
        def _init():
            acc_s[...] = jnp.zeros((1, 2), jnp.float32)

    @pl.when(p == 1)
    def _compute():
        fvh = fvh_s[...]
        fih = fih_s[...]
        avh = fvh_s[pl.ds(i * BLK, BLK), :]
        aih = fih_s[pl.ds(i * BLK, BLK), :]

        dn = (((1,), (1,)), ((), ()))

        def dot(a, b):
            return jax.lax.dot_general(
                a, b, dn, preferred_element_type=jnp.float32)

        simv = dot(avh, fvh)
        simi = dot(aih, fih)

        lab_row = labr_ref[...]
        lab_col = labc_ref[...]
        j_idx = jax.lax.broadcasted_iota(jnp.int32, (BLK, B), 1)
        row_ids = i * BLK + jax.lax.broadcasted_iota(jnp.int32, (BLK, B), 0)
        mask = (lab_col == lab_row) & (j_idx != row_ids)
        rev_j = 1023 - j_idx
        imin = jnp.int32(-2147483648)

        def kth_largest(work, k):
            for _ in range(k - 1):
                m = jnp.max(work, axis=1, keepdims=True)
                work = jnp.where(work == m, imin, work)
            return jnp.max(work, axis=1, keepdims=True)

        def sim_key(sim):
            bits = jax.lax.bitcast_convert_type(sim, jnp.int32)
            skey = bits ^ (jax.lax.shift_right_arithmetic(bits, 31)
                           & jnp.int32(0x7FFFFFFF))
            return jnp.where(mask, (skey & jnp.int32(~1023)) | rev_j, imin)

        keyv = sim_key(simv)
        keyi = sim_key(simi)
        x = dot(avh, fih)
        y = dot(aih, fvh)
        thr2 = kth_largest(jnp.concatenate([keyv, keyi], axis=0), K2)
        inter = ((keyv >= thr2[:BLK]) & (keyi >= thr2[BLK:])) & mask

        ikey = jnp.where(inter, rev_j, -1)
        thr4 = kth_largest(ikey, KC)
        m4f = ((ikey >= thr4) & inter).astype(jnp.float32)
        interf = inter.astype(jnp.float32)

        num = jnp.sum(m4f * x * y, axis=1, keepdims=True)
        na2 = jnp.sum(m4f * x * x, axis=1, keepdims=True)
        nb2 = jnp.sum(m4f * y * y, axis=1, keepdims=True)
        na = jnp.maximum(jnp.sqrt(na2), 1e-8)
        nb = jnp.maximum(jnp.sqrt(nb2), 1e-8)
        contrib = 1.0 - num / (na * nb)
        hasf = (jnp.sum(interf, axis=1, keepdims=True)
                > 0.0).astype(jnp.float32)

        part = jnp.concatenate(
            [jnp.sum(hasf * contrib, axis=0, keepdims=True),
             jnp.sum(hasf, axis=0, keepdims=True)], axis=1)
        acc_s[...] = acc_s[...] + part

        @pl.when(i == NBLK - 1)
        def _finish():
            total = acc_s[0:1, 0:1]
            cnt = acc_s[0:1, 1:2]
            loss = jnp.where(cnt == 0.0, 0.0, total / jnp.maximum(cnt, 1.0))
            loss = jnp.where(jnp.isnan(loss) | jnp.isinf(loss), 0.0, loss)
            out_ref[...] = jnp.maximum(loss, 0.0)


def _run(feat_v, feat_i, lab_row, lab_col, interpret=False):
    return pl.pallas_call(
        _graph_loss_kernel,
        grid=(2, NBLK),
        in_specs=[
            pl.BlockSpec((BLK, D), lambda p, i: (i, 0)),
            pl.BlockSpec((BLK, D), lambda p, i: (i, 0)),
            pl.BlockSpec((1, B), lambda p, i: (0, 0)),
            pl.BlockSpec((BLK, 1), lambda p, i: (i, 0)),
        ],
        out_specs=pl.BlockSpec((1, 1), lambda p, i: (0, 0)),
        out_shape=jax.ShapeDtypeStruct((1, 1), jnp.float32),
        scratch_shapes=[
            pltpu.VMEM((B, D), jnp.bfloat16),
            pltpu.VMEM((B, D), jnp.bfloat16),
            pltpu.VMEM((1, 2), jnp.float32),
        ],
        interpret=interpret,
    )(feat_v, feat_i, lab_row, lab_col)


@jax.jit
def kernel(feat_v, feat_i, labels):
    lab = labels.astype(jnp.int32)
    out = _run(feat_v.astype(jnp.float32), feat_i.astype(jnp.float32),
               lab.reshape(1, B), lab.reshape(B, 1))
    return out[0, 0]


# speedup vs baseline: 1.6177x; 1.0117x over previous
import jax
import jax.numpy as jnp
from jax.experimental import pallas as pl
from jax.experimental.pallas import tpu as pltpu

B = 1024
D = 2048
NBLK = 2
BLK = B // NBLK
K2 = 16
KC = 4
NEG_INF = -65000.0


def _graph_loss_kernel(fv_ref, fi_ref, labr_ref, labc_ref, out_ref,
                       fvh_s, fih_s, acc_s):
    s = pl.program_id(0)
    i = s - 1

    @pl.when(s == 0)
    def _normalize():
        fv = fv_ref[...]
        nv = jnp.sqrt(jnp.sum(fv * fv, axis=1, keepdims=True))
        fvh_s[...] = (fv / jnp.maximum(nv, 1e-12)).astype(jnp.bfloat16)
        fi = fi_ref[...]
        ni = jnp.sqrt(jnp.sum(fi * fi, axis=1, keepdims=True))
        fih_s[...] = (fi / jnp.maximum(ni, 1e-12)).astype(jnp.bfloat16)
        acc_s[...] = jnp.zeros((1, 2), jnp.float32)

    @pl.when(s > 0)
    def _compute():
        fvh = fvh_s[...]
        fih = fih_s[...]
        avh = fvh_s[pl.ds(i * BLK, BLK), :]
        aih = fih_s[pl.ds(i * BLK, BLK), :]

        dn = (((1,), (1,)), ((), ()))

        def dot(a, b):
            return jax.lax.dot_general(
                a, b, dn, preferred_element_type=jnp.float32)

        simv = dot(avh, fvh)
        simi = dot(aih, fih)

        lab_row = labr_ref[...]
        lab_col = labc_ref[...]
        j_idx = jax.lax.broadcasted_iota(jnp.int32, (BLK, B), 1)
        row_ids = i * BLK + jax.lax.broadcasted_iota(jnp.int32, (BLK, B), 0)
        mask = (lab_col == lab_row) & (j_idx != row_ids)
        rev_j = 1023 - j_idx
        imin = jnp.int32(-2147483648)

        def kth_largest(work, k):
            for _ in range(k - 1):
                m = jnp.max(work, axis=1, keepdims=True)
                work = jnp.where(work == m, imin, work)
            return jnp.max(work, axis=1, keepdims=True)

        def sim_key(sim):
            bits = jax.lax.bitcast_convert_type(sim, jnp.int32)
            skey = bits ^ (jax.lax.shift_right_arithmetic(bits, 31)
                           & jnp.int32(0x7FFFFFFF))
            return jnp.where(mask, (skey & jnp.int32(~1023)) | rev_j, imin)

        keyv = sim_key(simv)
        keyi = sim_key(simi)
        x = dot(avh, fih)
        y = dot(aih, fvh)
        thr2 = kth_largest(jnp.concatenate([keyv, keyi], axis=0), K2)
        inter = ((keyv >= thr2[:BLK]) & (keyi >= thr2[BLK:])) & mask

        ikey = jnp.where(inter, rev_j, -1)
        thr4 = kth_largest(ikey, KC)
        m4f = ((ikey >= thr4) & inter).astype(jnp.float32)
        interf = inter.astype(jnp.float32)

        num = jnp.sum(m4f * x * y, axis=1, keepdims=True)
        na2 = jnp.sum(m4f * x * x, axis=1, keepdims=True)
        nb2 = jnp.sum(m4f * y * y, axis=1, keepdims=True)
        na = jnp.maximum(jnp.sqrt(na2), 1e-8)
        nb = jnp.maximum(jnp.sqrt(nb2), 1e-8)
        contrib = 1.0 - num / (na * nb)
        hasf = (jnp.sum(interf, axis=1, keepdims=True)
                > 0.0).astype(jnp.float32)

        part = jnp.concatenate(
            [jnp.sum(hasf * contrib, axis=0, keepdims=True),
             jnp.sum(hasf, axis=0, keepdims=True)], axis=1)
        acc_s[...] = acc_s[...] + part

        @pl.when(i == NBLK - 1)
        def _finish():
            total = acc_s[0:1, 0:1]
            cnt = acc_s[0:1, 1:2]
            loss = jnp.where(cnt == 0.0, 0.0, total / jnp.maximum(cnt, 1.0))
            loss = jnp.where(jnp.isnan(loss) | jnp.isinf(loss), 0.0, loss)
            out_ref[...] = jnp.maximum(loss, 0.0)


def _run(feat_v, feat_i, lab_row, lab_col, interpret=False):
    return pl.pallas_call(
        _graph_loss_kernel,
        grid=(NBLK + 1,),
        in_specs=[
            pl.BlockSpec((B, D), lambda s: (0, 0)),
            pl.BlockSpec((B, D), lambda s: (0, 0)),
            pl.BlockSpec((1, B), lambda s: (0, 0)),
            pl.BlockSpec((BLK, 1), lambda s: (jnp.maximum(s - 1, 0), 0)),
        ],
        out_specs=pl.BlockSpec((1, 1), lambda s: (0, 0)),
        out_shape=jax.ShapeDtypeStruct((1, 1), jnp.float32),
        scratch_shapes=[
            pltpu.VMEM((B, D), jnp.bfloat16),
            pltpu.VMEM((B, D), jnp.bfloat16),
            pltpu.VMEM((1, 2), jnp.float32),
        ],
        interpret=interpret,
    )(feat_v, feat_i, lab_row, lab_col)


@jax.jit
def kernel(feat_v, feat_i, labels):
    lab = labels.astype(jnp.int32)
    out = _run(feat_v.astype(jnp.float32), feat_i.astype(jnp.float32),
               lab.reshape(1, B), lab.reshape(B, 1))
    return out[0, 0]
